# baseline (speedup 1.0000x reference)
import functools

import jax
import jax.numpy as jnp
from jax import lax
from jax.experimental import pallas as pl
from jax.experimental.pallas import tpu as pltpu
from jax.experimental.pallas import tpu_sc as plsc

N_NODES = 10000
N_EDGES = 320000
F = 128
LAT = 64

NC = 2
NS = 16
CHUNK = 80
CPT = 128
E_PAD = NC * NS * CPT * CHUNK
CROWS = E_PAD // CHUNK
NROWS = N_NODES + 8
DEGN = 10240

_MESH = plsc.VectorSubcoreMesh(core_axis_name="c", subcore_axis_name="s")


def _sc_degree_body(dst2, zn, degp, dstall, onesv, shared):
    c = lax.axis_index("c")
    s = lax.axis_index("s")
    base = (c * NS + s) * CPT

    @pl.when(s == 0)
    def _():
        pltpu.sync_copy(zn, shared)

    pltpu.sync_copy(dst2.at[pl.ds(base, CPT)], dstall)

    def fill(i, _):
        onesv[pl.ds(i * 16, 16)] = jnp.ones((16,), jnp.float32)
        return 0

    lax.fori_loop(0, CHUNK // 16, fill, 0)
    plsc.subcore_barrier()

    def chunk(j, _):
        pltpu.sync_copy(onesv, shared.at[dstall.at[j]], add=True)
        return 0

    lax.fori_loop(0, CPT, chunk, 0)
    plsc.subcore_barrier()

    @pl.when(s == 0)
    def _():
        pltpu.sync_copy(shared, degp.at[c])


_sc_degree = pl.kernel(
    _sc_degree_body,
    out_type=jax.ShapeDtypeStruct((NC, DEGN), jnp.float32),
    mesh=_MESH,
    scratch_types=[
        pltpu.VMEM((CPT, CHUNK), jnp.int32),
        pltpu.VMEM((CHUNK,), jnp.float32),
        pltpu.VMEM_SHARED((DEGN,), jnp.float32),
    ],
)


NBUF = 2


def _sc_prop_body(g, comb, zz, accp, comball, idxb, rowsb, accum, gsems, ssems):
    c = lax.axis_index("c")
    s = lax.axis_index("s")
    base = (c * NS + s) * CPT

    @pl.when(s == 0)
    def _():
        pltpu.sync_copy(zz, accum)

    pltpu.sync_copy(comb.at[pl.ds(base, CPT)], comball)
    plsc.subcore_barrier()

    def unpack(r, k):
        for i in range(CHUNK // 16):
            v = comball[r, pl.ds(i * 16, 16)]
            idxb[k, pl.ds(i * 16, 16)] = lax.bitwise_and(v, 0xFFFF)
            idxb[NBUF + k, pl.ds(i * 16, 16)] = lax.shift_right_logical(v, 16)

    for k in range(NBUF):
        unpack(k, k)
        pltpu.async_copy(g.at[idxb.at[k]], rowsb.at[k], gsems.at[k])

    def block(jj, _):
        j = jj * NBUF
        for k in range(NBUF):
            pltpu.make_async_copy(g.at[idxb.at[k]], rowsb.at[k],
                                  gsems.at[k]).wait()
            pltpu.async_copy(rowsb.at[k], accum.at[idxb.at[NBUF + k]],
                             ssems.at[k], add=True)

        @pl.when(jj + 1 < CPT // NBUF)
        def _():
            for k in range(NBUF):
                pltpu.make_async_copy(rowsb.at[k], accum.at[idxb.at[NBUF + k]],
                                      ssems.at[k]).wait()
                unpack(j + NBUF + k, k)
                pltpu.async_copy(g.at[idxb.at[k]], rowsb.at[k], gsems.at[k])
        return 0

    lax.fori_loop(0, CPT // NBUF, block, 0)
    for k in range(NBUF):
        pltpu.make_async_copy(rowsb.at[k], accum.at[idxb.at[NBUF + k]],
                              ssems.at[k]).wait()
    plsc.subcore_barrier()

    @pl.when(s == 0)
    def _():
        pltpu.sync_copy(accum.at[pl.ds(0, N_NODES)], accp.at[c])


_sc_prop = pl.kernel(
    _sc_prop_body,
    out_type=jax.ShapeDtypeStruct((NC, N_NODES, F), jnp.float32),
    mesh=_MESH,
    scratch_types=[
        pltpu.VMEM((CPT, CHUNK), jnp.int32),
        pltpu.VMEM((2 * NBUF, CHUNK), jnp.int32),
        pltpu.VMEM((NBUF, CHUNK, F), jnp.float32),
        pltpu.VMEM_SHARED((NROWS, F), jnp.float32),
        pltpu.SemaphoreType.DMA((NBUF,)),
        pltpu.SemaphoreType.DMA((NBUF,)),
    ],
)


_BLK = 1000
_GRID = N_NODES // _BLK


def _tc_prep_body(x_ref, y_ref, dg_ref, w1_ref, g1_ref, dinv_ref):
    deg = dg_ref[:, 0:1] + dg_ref[:, 1:2] + 1.0
    dinv = lax.rsqrt(deg)
    hw = jnp.dot(jnp.abs(x_ref[...]), w1_ref[0:F, :],
                 preferred_element_type=jnp.float32)
    hw = hw + jnp.abs(y_ref[...]) * w1_ref[F:F + 1, :]
    g1_ref[...] = dinv * hw
    dinv_ref[...] = dinv


_tc_prep = pl.pallas_call(
    _tc_prep_body,
    grid=(_GRID,),
    in_specs=[
        pl.BlockSpec((_BLK, F), lambda i: (i, 0)),
        pl.BlockSpec((_BLK, 1), lambda i: (i, 0)),
        pl.BlockSpec((_BLK, NC), lambda i: (i, 0)),
        pl.BlockSpec((F + 1, F), lambda i: (0, 0)),
    ],
    out_specs=[
        pl.BlockSpec((_BLK, F), lambda i: (i, 0)),
        pl.BlockSpec((_BLK, 1), lambda i: (i, 0)),
    ],
    out_shape=[
        jax.ShapeDtypeStruct((N_NODES, F), jnp.float32),
        jax.ShapeDtypeStruct((N_NODES, 1), jnp.float32),
    ],
)


def _tc_mid_body(acc_ref, g1_ref, dinv_ref, b1_ref, wcat_ref, g2_ref):
    dinv = dinv_ref[...]
    acc = acc_ref[0] + acc_ref[1] + g1_ref[...]
    h = jnp.maximum(dinv * acc + b1_ref[...], 0.0)
    g2_ref[...] = dinv * jnp.dot(h, wcat_ref[...],
                                 preferred_element_type=jnp.float32)


_tc_mid = pl.pallas_call(
    _tc_mid_body,
    grid=(_GRID,),
    in_specs=[
        pl.BlockSpec((NC, _BLK, F), lambda i: (0, i, 0)),
        pl.BlockSpec((_BLK, F), lambda i: (i, 0)),
        pl.BlockSpec((_BLK, 1), lambda i: (i, 0)),
        pl.BlockSpec((1, F), lambda i: (0, 0)),
        pl.BlockSpec((F, F), lambda i: (0, 0)),
    ],
    out_specs=pl.BlockSpec((_BLK, F), lambda i: (i, 0)),
    out_shape=jax.ShapeDtypeStruct((N_NODES, F), jnp.float32),
)


def _tc_head_body(acc_ref, g2_ref, dinv_ref, bmu_ref, bls_ref, mu_ref, std_ref):
    out = dinv_ref[...] * (acc_ref[0] + acc_ref[1] + g2_ref[...])
    mu_ref[...] = out[:, 0:LAT] + bmu_ref[...]
    z = out[:, LAT:2 * LAT] + bls_ref[...]
    std_ref[...] = jnp.maximum(z, 0.0) + jnp.log1p(jnp.exp(-jnp.abs(z)))


_tc_head = pl.pallas_call(
    _tc_head_body,
    grid=(_GRID,),
    in_specs=[
        pl.BlockSpec((NC, _BLK, F), lambda i: (0, i, 0)),
        pl.BlockSpec((_BLK, F), lambda i: (i, 0)),
        pl.BlockSpec((_BLK, 1), lambda i: (i, 0)),
        pl.BlockSpec((1, LAT), lambda i: (0, 0)),
        pl.BlockSpec((1, LAT), lambda i: (0, 0)),
    ],
    out_specs=[
        pl.BlockSpec((_BLK, LAT), lambda i: (i, 0)),
        pl.BlockSpec((_BLK, LAT), lambda i: (i, 0)),
    ],
    out_shape=[
        jax.ShapeDtypeStruct((N_NODES, LAT), jnp.float32),
        jax.ShapeDtypeStruct((N_NODES, LAT), jnp.float32),
    ],
)


@jax.jit
def kernel(x, edge_index, Y, W1, b1, Wmu, bmu, Wls, bls):
    ei = edge_index.astype(jnp.int32)
    npad = E_PAD - N_EDGES
    srcp = jnp.concatenate([ei[0], jnp.zeros((npad,), jnp.int32)])
    dstp = jnp.concatenate([ei[1], jnp.full((npad,), N_NODES, jnp.int32)])
    dst2 = dstp.reshape(CROWS, CHUNK)
    comb = (srcp | (dstp << 16)).reshape(CROWS, CHUNK)
    zn = jnp.zeros((DEGN,), jnp.float32)
    zz = jnp.zeros((NROWS, F), jnp.float32)

    degp = _sc_degree(dst2, zn)
    degt = degp[:, :N_NODES].T

    g1, dinv = _tc_prep(x, Y, degt, W1)
    acc1 = _sc_prop(g1, comb, zz)
    wcat = jnp.concatenate([Wmu, Wls], axis=1)
    g2 = _tc_mid(acc1, g1, dinv, b1.reshape(1, F), wcat)
    acc2 = _sc_prop(g2, comb, zz)
    mu, std = _tc_head(acc2, g2, dinv, bmu.reshape(1, LAT), bls.reshape(1, LAT))
    return (mu, std)


# speedup vs baseline: 2.8081x; 2.8081x over previous
import functools

import jax
import jax.numpy as jnp
from jax import lax
from jax.experimental import pallas as pl
from jax.experimental.pallas import tpu as pltpu
from jax.experimental.pallas import tpu_sc as plsc

N_NODES = 10000
N_EDGES = 320000
F = 128
LAT = 64

NC = 2
NS = 16
CHUNK = 80
CPT = 128
E_PAD = NC * NS * CPT * CHUNK
CROWS = E_PAD // CHUNK
NROWS = 10240
NDUMP = NROWS - N_NODES
DEGN = 10240

_MESH = plsc.VectorSubcoreMesh(core_axis_name="c", subcore_axis_name="s")


def _sc_degree_body(dst2, zn, degp, dstall, onesv, shared):
    c = lax.axis_index("c")
    s = lax.axis_index("s")
    base = (c * NS + s) * CPT

    @pl.when(s == 0)
    def _():
        pltpu.sync_copy(zn, shared)

    pltpu.sync_copy(dst2.at[pl.ds(base, CPT)], dstall)

    def fill(i, _):
        onesv[pl.ds(i * 16, 16)] = jnp.ones((16,), jnp.float32)
        return 0

    lax.fori_loop(0, CHUNK // 16, fill, 0)
    plsc.subcore_barrier()

    def chunk(j, _):
        pltpu.sync_copy(onesv, shared.at[dstall.at[j]], add=True)
        return 0

    lax.fori_loop(0, CPT, chunk, 0)
    plsc.subcore_barrier()

    @pl.when(s == 0)
    def _():
        pltpu.sync_copy(shared, degp.at[c])


_sc_degree = pl.kernel(
    _sc_degree_body,
    out_type=jax.ShapeDtypeStruct((NC, DEGN), jnp.float32),
    mesh=_MESH,
    scratch_types=[
        pltpu.VMEM((CPT, CHUNK), jnp.int32),
        pltpu.VMEM((CHUNK,), jnp.float32),
        pltpu.VMEM_SHARED((DEGN,), jnp.float32),
    ],
)


NBUF = 2


def _sc_prop_body(g, comb, accp, comball, idxb, rowsb, accum, gsems, ssems):
    c = lax.axis_index("c")
    s = lax.axis_index("s")
    base = (c * NS + s) * CPT

    pltpu.sync_copy(comb.at[pl.ds(base, CPT)], comball)

    def zrow(i, _):
        rowsb[0, i // 8, pl.ds((i % 8) * 16, 16)] = jnp.zeros((16,), jnp.float32)
        return 0

    lax.fori_loop(0, CHUNK * 8, zrow, 0)
    zslice = NROWS // NS
    for i in range(zslice // CHUNK):
        pltpu.sync_copy(rowsb.at[0],
                        accum.at[pl.ds(s * zslice + i * CHUNK, CHUNK)])
    plsc.subcore_barrier()

    def unpack(r, k):
        for i in range(CHUNK // 16):
            v = comball[r, pl.ds(i * 16, 16)]
            idxb[k, pl.ds(i * 16, 16)] = lax.bitwise_and(v, 0xFFFF)
            idxb[NBUF + k, pl.ds(i * 16, 16)] = lax.shift_right_logical(v, 16)

    for k in range(NBUF):
        unpack(k, k)
        pltpu.async_copy(g.at[idxb.at[k]], rowsb.at[k], gsems.at[k])

    def block(jj, _):
        j = jj * NBUF
        for k in range(NBUF):
            pltpu.make_async_copy(g.at[idxb.at[k]], rowsb.at[k],
                                  gsems.at[k]).wait()
            pltpu.async_copy(rowsb.at[k], accum.at[idxb.at[NBUF + k]],
                             ssems.at[k], add=True)

        @pl.when(jj + 1 < CPT // NBUF)
        def _():
            for k in range(NBUF):
                pltpu.make_async_copy(rowsb.at[k], accum.at[idxb.at[NBUF + k]],
                                      ssems.at[k]).wait()
                unpack(j + NBUF + k, k)
                pltpu.async_copy(g.at[idxb.at[k]], rowsb.at[k], gsems.at[k])
        return 0

    lax.fori_loop(0, CPT // NBUF, block, 0)
    for k in range(NBUF):
        pltpu.make_async_copy(rowsb.at[k], accum.at[idxb.at[NBUF + k]],
                              ssems.at[k]).wait()
    plsc.subcore_barrier()

    wslice = 1000
    @pl.when(s < 10)
    def _():
        pltpu.sync_copy(accum.at[pl.ds(s * wslice, wslice)],
                        accp.at[c, pl.ds(s * wslice, wslice)])


_sc_prop = pl.kernel(
    _sc_prop_body,
    out_type=jax.ShapeDtypeStruct((NC, N_NODES, F), jnp.float32),
    mesh=_MESH,
    scratch_types=[
        pltpu.VMEM((CPT, CHUNK), jnp.int32),
        pltpu.VMEM((2 * NBUF, CHUNK), jnp.int32),
        pltpu.VMEM((NBUF, CHUNK, F), jnp.float32),
        pltpu.VMEM_SHARED((NROWS, F), jnp.float32),
        pltpu.SemaphoreType.DMA((NBUF,)),
        pltpu.SemaphoreType.DMA((NBUF,)),
    ],
)


_BLK = 1000
_GRID = N_NODES // _BLK


def _tc_prep_body(x_ref, y_ref, dg_ref, w1_ref, g1_ref, dinv_ref):
    deg = dg_ref[:, 0:1] + dg_ref[:, 1:2] + 1.0
    dinv = lax.rsqrt(deg)
    hw = jnp.dot(jnp.abs(x_ref[...]), w1_ref[0:F, :],
                 preferred_element_type=jnp.float32)
    hw = hw + jnp.abs(y_ref[...]) * w1_ref[F:F + 1, :]
    g1_ref[...] = dinv * hw
    dinv_ref[...] = dinv


_tc_prep = pl.pallas_call(
    _tc_prep_body,
    grid=(_GRID,),
    in_specs=[
        pl.BlockSpec((_BLK, F), lambda i: (i, 0)),
        pl.BlockSpec((_BLK, 1), lambda i: (i, 0)),
        pl.BlockSpec((_BLK, NC), lambda i: (i, 0)),
        pl.BlockSpec((F + 1, F), lambda i: (0, 0)),
    ],
    out_specs=[
        pl.BlockSpec((_BLK, F), lambda i: (i, 0)),
        pl.BlockSpec((_BLK, 1), lambda i: (i, 0)),
    ],
    out_shape=[
        jax.ShapeDtypeStruct((N_NODES, F), jnp.float32),
        jax.ShapeDtypeStruct((N_NODES, 1), jnp.float32),
    ],
)


def _tc_mid_body(acc_ref, g1_ref, dinv_ref, b1_ref, wcat_ref, g2_ref):
    dinv = dinv_ref[...]
    acc = acc_ref[0] + acc_ref[1] + g1_ref[...]
    h = jnp.maximum(dinv * acc + b1_ref[...], 0.0)
    g2_ref[...] = dinv * jnp.dot(h, wcat_ref[...],
                                 preferred_element_type=jnp.float32)


_tc_mid = pl.pallas_call(
    _tc_mid_body,
    grid=(_GRID,),
    in_specs=[
        pl.BlockSpec((NC, _BLK, F), lambda i: (0, i, 0)),
        pl.BlockSpec((_BLK, F), lambda i: (i, 0)),
        pl.BlockSpec((_BLK, 1), lambda i: (i, 0)),
        pl.BlockSpec((1, F), lambda i: (0, 0)),
        pl.BlockSpec((F, F), lambda i: (0, 0)),
    ],
    out_specs=pl.BlockSpec((_BLK, F), lambda i: (i, 0)),
    out_shape=jax.ShapeDtypeStruct((N_NODES, F), jnp.float32),
)


def _tc_head_body(acc_ref, g2_ref, dinv_ref, bmu_ref, bls_ref, mu_ref, std_ref):
    out = dinv_ref[...] * (acc_ref[0] + acc_ref[1] + g2_ref[...])
    mu_ref[...] = out[:, 0:LAT] + bmu_ref[...]
    z = out[:, LAT:2 * LAT] + bls_ref[...]
    std_ref[...] = jnp.maximum(z, 0.0) + jnp.log1p(jnp.exp(-jnp.abs(z)))


_tc_head = pl.pallas_call(
    _tc_head_body,
    grid=(_GRID,),
    in_specs=[
        pl.BlockSpec((NC, _BLK, F), lambda i: (0, i, 0)),
        pl.BlockSpec((_BLK, F), lambda i: (i, 0)),
        pl.BlockSpec((_BLK, 1), lambda i: (i, 0)),
        pl.BlockSpec((1, LAT), lambda i: (0, 0)),
        pl.BlockSpec((1, LAT), lambda i: (0, 0)),
    ],
    out_specs=[
        pl.BlockSpec((_BLK, LAT), lambda i: (i, 0)),
        pl.BlockSpec((_BLK, LAT), lambda i: (i, 0)),
    ],
    out_shape=[
        jax.ShapeDtypeStruct((N_NODES, LAT), jnp.float32),
        jax.ShapeDtypeStruct((N_NODES, LAT), jnp.float32),
    ],
)


@jax.jit
def kernel(x, edge_index, Y, W1, b1, Wmu, bmu, Wls, bls):
    ei = edge_index.astype(jnp.int32)
    npad = E_PAD - N_EDGES
    pad = jnp.arange(npad, dtype=jnp.int32)
    srcp = jnp.concatenate([ei[0], pad % N_NODES])
    dstp = jnp.concatenate([ei[1], N_NODES + pad % NDUMP])
    dst2 = dstp.reshape(CROWS, CHUNK)
    comb = (srcp | (dstp << 16)).reshape(CROWS, CHUNK)
    zn = jnp.zeros((DEGN,), jnp.float32)

    degp = _sc_degree(dst2, zn)
    degt = degp[:, :N_NODES].T

    g1, dinv = _tc_prep(x, Y, degt, W1)
    acc1 = _sc_prop(g1, comb)
    wcat = jnp.concatenate([Wmu, Wls], axis=1)
    g2 = _tc_mid(acc1, g1, dinv, b1.reshape(1, F), wcat)
    acc2 = _sc_prop(g2, comb)
    mu, std = _tc_head(acc2, g2, dinv, bmu.reshape(1, LAT), bls.reshape(1, LAT))
    return (mu, std)
